# Initial kernel scaffold; baseline (speedup 1.0000x reference)
import jax
import jax.numpy as jnp
from jax import lax
from jax.experimental import pallas as pl
from jax.experimental.pallas import tpu as pltpu

TOPK_K = 64


def _fused_body(mem_ref, keys_ref, beta_ref, wm_ref, bm_ref, gam_ref,
                lnb_ref, out_ref, w_ref):
    mem = mem_ref[0]
    keys = keys_ref[0]
    beta = beta_ref[0]
    H = keys.shape[0]
    N, D = mem.shape

    knorm = jnp.sqrt(jnp.sum(keys * keys, axis=1, keepdims=True))
    kn = keys / jnp.maximum(knorm, 1e-12)

    sim_raw = lax.dot_general(kn, mem, (((1,), (1,)), ((), ())),
                              preferred_element_type=jnp.float32)
    sq = lax.dot_general(jnp.ones((8, D), jnp.float32), mem * mem,
                         (((1,), (1,)), ((), ())),
                         preferred_element_type=jnp.float32)[:1]
    scale = 1.0 / jnp.maximum(jnp.sqrt(sq), 1e-12)
    sim = sim_raw * scale * beta

    bits = lax.bitcast_convert_type(sim, jnp.uint32)
    sign = bits >> 31
    ordered = bits ^ (jnp.uint32(0x80000000) + sign * jnp.uint32(0x7FFFFFFF))

    def step(i, cand):
        test = cand | (jnp.uint32(1) << (31 - i).astype(jnp.uint32))
        cnt = jnp.sum((ordered >= test).astype(jnp.int32), axis=1,
                      keepdims=True)
        return jnp.where(cnt >= TOPK_K, test, cand)

    cand = lax.fori_loop(0, 32, step, jnp.zeros((H, 1), jnp.uint32))
    mask = ordered >= cand

    rowmax = jnp.max(sim, axis=1, keepdims=True)
    e = jnp.where(mask, jnp.exp(sim - rowmax), 0.0)
    w = e / jnp.sum(e, axis=1, keepdims=True)
    w_ref[0] = w

    rph = lax.dot_general(w, mem, (((1,), (0,)), ((), ())),
                          preferred_element_type=jnp.float32)

    merged = bm_ref[...]
    for h in range(H):
        merged = merged + lax.dot_general(
            rph[h:h + 1], wm_ref[h], (((1,), (1,)), ((), ())),
            preferred_element_type=jnp.float32)

    mu = jnp.mean(merged, axis=1, keepdims=True)
    var = jnp.mean((merged - mu) ** 2, axis=1, keepdims=True)
    out_ref[...] = ((merged - mu) / jnp.sqrt(var + 1e-5)) * gam_ref[...] \
        + lnb_ref[...]


@jax.jit
def kernel(memory, read_keys, beta, W_merge, b_merge, ln_gamma, ln_beta):
    B, N, D = memory.shape
    H = read_keys.shape[1]
    wm_r = W_merge.reshape(D, H, D).transpose(1, 0, 2)
    beta_r = beta[..., None]
    bm2 = b_merge[None, :]
    gam2 = ln_gamma[None, :]
    lnb2 = ln_beta[None, :]

    out = pl.pallas_call(
        _fused_body,
        grid=(B,),
        in_specs=[
            pl.BlockSpec((1, N, D), lambda b: (b, 0, 0)),
            pl.BlockSpec((1, H, D), lambda b: (b, 0, 0)),
            pl.BlockSpec((1, H, 1), lambda b: (b, 0, 0)),
            pl.BlockSpec((H, D, D), lambda b: (0, 0, 0)),
            pl.BlockSpec((1, D), lambda b: (0, 0)),
            pl.BlockSpec((1, D), lambda b: (0, 0)),
            pl.BlockSpec((1, D), lambda b: (0, 0)),
        ],
        out_specs=[
            pl.BlockSpec((1, D), lambda b: (b, 0)),
            pl.BlockSpec((1, H, N), lambda b: (b, 0, 0)),
        ],
        out_shape=[
            jax.ShapeDtypeStruct((B, D), jnp.float32),
            jax.ShapeDtypeStruct((B, H, N), jnp.float32),
        ],
        compiler_params=pltpu.CompilerParams(
            dimension_semantics=("arbitrary",),
            vmem_limit_bytes=100 * 1024 * 1024,
        ),
    )(memory, read_keys, beta_r, wm_r, bm2, gam2, lnb2)
    return out[0], out[1]


# speedup vs baseline: 2.3123x; 2.3123x over previous
import jax
import jax.numpy as jnp
from jax import lax
from jax.experimental import pallas as pl
from jax.experimental.pallas import tpu as pltpu

TOPK_K = 64


def _fused_body(mem_ref, keys_ref, beta_ref, wm_ref, bm_ref, gam_ref,
                lnb_ref, out_ref, w_ref):
    mem = mem_ref[0]
    keys = keys_ref[0]
    beta = beta_ref[0]
    H = keys.shape[0]
    N, D = mem.shape

    knorm = jnp.sqrt(jnp.sum(keys * keys, axis=1, keepdims=True))
    kn = keys / jnp.maximum(knorm, 1e-12)
    mnorm = jnp.sqrt(jnp.sum(mem * mem, axis=1, keepdims=True))
    mn = mem / jnp.maximum(mnorm, 1e-12)

    sim = lax.dot_general(kn, mn, (((1,), (1,)), ((), ())),
                          preferred_element_type=jnp.float32) * beta

    bits = lax.bitcast_convert_type(sim, jnp.uint32)
    sign = bits >> 31
    ordered = bits ^ (jnp.uint32(0x80000000) + sign * jnp.uint32(0x7FFFFFFF))

    def step(i, cand):
        test = cand | (jnp.uint32(1) << (31 - i).astype(jnp.uint32))
        cnt = jnp.sum((ordered >= test).astype(jnp.int32), axis=1,
                      keepdims=True)
        return jnp.where(cnt >= TOPK_K, test, cand)

    cand = lax.fori_loop(0, 32, step, jnp.zeros((H, 1), jnp.uint32))
    mask = ordered >= cand

    rowmax = jnp.max(sim, axis=1, keepdims=True)
    e = jnp.where(mask, jnp.exp(sim - rowmax), 0.0)
    w = e / jnp.sum(e, axis=1, keepdims=True)
    w_ref[0] = w

    rph = lax.dot_general(w, mem, (((1,), (0,)), ((), ())),
                          preferred_element_type=jnp.float32, precision=lax.Precision.HIGHEST)

    merged = bm_ref[...]
    for h in range(H):
        merged = merged + lax.dot_general(
            rph[h:h + 1], wm_ref[h], (((1,), (1,)), ((), ())),
            preferred_element_type=jnp.float32, precision=lax.Precision.HIGHEST)

    mu = jnp.mean(merged, axis=1, keepdims=True)
    var = jnp.mean((merged - mu) ** 2, axis=1, keepdims=True)
    out_ref[0] = ((merged - mu) / jnp.sqrt(var + 1e-5)) * gam_ref[...] \
        + lnb_ref[...]


@jax.jit
def kernel(memory, read_keys, beta, W_merge, b_merge, ln_gamma, ln_beta):
    B, N, D = memory.shape
    H = read_keys.shape[1]
    wm_r = W_merge.reshape(D, H, D).transpose(1, 0, 2)
    beta_r = beta[..., None]
    bm2 = b_merge[None, :]
    gam2 = ln_gamma[None, :]
    lnb2 = ln_beta[None, :]

    out = pl.pallas_call(
        _fused_body,
        grid=(B,),
        in_specs=[
            pl.BlockSpec((1, N, D), lambda b: (b, 0, 0)),
            pl.BlockSpec((1, H, D), lambda b: (b, 0, 0)),
            pl.BlockSpec((1, H, 1), lambda b: (b, 0, 0)),
            pl.BlockSpec((H, D, D), lambda b: (0, 0, 0)),
            pl.BlockSpec((1, D), lambda b: (0, 0)),
            pl.BlockSpec((1, D), lambda b: (0, 0)),
            pl.BlockSpec((1, D), lambda b: (0, 0)),
        ],
        out_specs=[
            pl.BlockSpec((1, 1, D), lambda b: (b, 0, 0)),
            pl.BlockSpec((1, H, N), lambda b: (b, 0, 0)),
        ],
        out_shape=[
            jax.ShapeDtypeStruct((B, 1, D), jnp.float32),
            jax.ShapeDtypeStruct((B, H, N), jnp.float32),
        ],
        compiler_params=pltpu.CompilerParams(
            dimension_semantics=("arbitrary",),
            vmem_limit_bytes=100 * 1024 * 1024,
        ),
    )(memory, read_keys, beta_r, wm_r, bm2, gam2, lnb2)
    return out[0][:, 0, :], out[1]


# speedup vs baseline: 4.4081x; 1.9063x over previous
import jax
import jax.numpy as jnp
from jax import lax
from jax.experimental import pallas as pl
from jax.experimental.pallas import tpu as pltpu

TOPK_K = 64


def _fused_body(mem_ref, keys_ref, beta_ref, wm_ref, bm_ref, gam_ref,
                lnb_ref, out_ref, w_ref):
    mem = mem_ref[0]
    keys = keys_ref[0]
    beta = beta_ref[0]
    H = keys.shape[0]
    N, D = mem.shape

    knorm = jnp.sqrt(jnp.sum(keys * keys, axis=1, keepdims=True))
    kn = keys / jnp.maximum(knorm, 1e-12)
    mnorm = jnp.sqrt(jnp.sum(mem * mem, axis=1, keepdims=True))
    mn = mem / jnp.maximum(mnorm, 1e-12)

    sim = lax.dot_general(kn, mn, (((1,), (1,)), ((), ())),
                          preferred_element_type=jnp.float32) * beta

    bits = lax.bitcast_convert_type(sim, jnp.uint32)
    sign = bits >> 31
    ordered = bits ^ (jnp.uint32(0x80000000) + sign * jnp.uint32(0x7FFFFFFF))

    def step(i, cand):
        test = cand | (jnp.uint32(1) << (31 - i).astype(jnp.uint32))
        cnt = jnp.sum((ordered >= test).astype(jnp.int32), axis=1,
                      keepdims=True)
        return jnp.where(cnt >= TOPK_K, test, cand)

    cand = lax.fori_loop(0, 32, step, jnp.zeros((H, 1), jnp.uint32))
    mask = ordered >= cand

    rowmax = jnp.max(sim, axis=1, keepdims=True)
    e = jnp.where(mask, jnp.exp(sim - rowmax), 0.0)
    w = e / jnp.sum(e, axis=1, keepdims=True)
    w_ref[0] = w

    rph = lax.dot_general(w, mem, (((1,), (0,)), ((), ())),
                          preferred_element_type=jnp.float32)

    merged = bm_ref[...]
    for h in range(H):
        merged = merged + lax.dot_general(
            rph[h:h + 1], wm_ref[h], (((1,), (1,)), ((), ())),
            preferred_element_type=jnp.float32)

    mu = jnp.mean(merged, axis=1, keepdims=True)
    var = jnp.mean((merged - mu) ** 2, axis=1, keepdims=True)
    out_ref[0] = ((merged - mu) / jnp.sqrt(var + 1e-5)) * gam_ref[...] \
        + lnb_ref[...]


@jax.jit
def kernel(memory, read_keys, beta, W_merge, b_merge, ln_gamma, ln_beta):
    B, N, D = memory.shape
    H = read_keys.shape[1]
    wm_r = W_merge.reshape(D, H, D).transpose(1, 0, 2)
    beta_r = beta[..., None]
    bm2 = b_merge[None, :]
    gam2 = ln_gamma[None, :]
    lnb2 = ln_beta[None, :]

    out = pl.pallas_call(
        _fused_body,
        grid=(B,),
        in_specs=[
            pl.BlockSpec((1, N, D), lambda b: (b, 0, 0)),
            pl.BlockSpec((1, H, D), lambda b: (b, 0, 0)),
            pl.BlockSpec((1, H, 1), lambda b: (b, 0, 0)),
            pl.BlockSpec((H, D, D), lambda b: (0, 0, 0)),
            pl.BlockSpec((1, D), lambda b: (0, 0)),
            pl.BlockSpec((1, D), lambda b: (0, 0)),
            pl.BlockSpec((1, D), lambda b: (0, 0)),
        ],
        out_specs=[
            pl.BlockSpec((1, 1, D), lambda b: (b, 0, 0)),
            pl.BlockSpec((1, H, N), lambda b: (b, 0, 0)),
        ],
        out_shape=[
            jax.ShapeDtypeStruct((B, 1, D), jnp.float32),
            jax.ShapeDtypeStruct((B, H, N), jnp.float32),
        ],
        compiler_params=pltpu.CompilerParams(
            dimension_semantics=("arbitrary",),
            vmem_limit_bytes=100 * 1024 * 1024,
        ),
    )(memory, read_keys, beta_r, wm_r, bm2, gam2, lnb2)
    return out[0][:, 0, :], out[1]


# speedup vs baseline: 4.5866x; 1.0405x over previous
import jax
import jax.numpy as jnp
from jax import lax
from jax.experimental import pallas as pl
from jax.experimental.pallas import tpu as pltpu

TOPK_K = 64


def _one_batch(i, mem_ref, keys_ref, beta_ref, wm_ref, bm_ref, gam_ref,
               lnb_ref, out_ref, w_ref):
    mem = mem_ref[i]
    keys = keys_ref[i]
    beta = beta_ref[i]
    H = keys.shape[0]
    N, D = mem.shape

    knorm = jnp.sqrt(jnp.sum(keys * keys, axis=1, keepdims=True))
    kn = keys / jnp.maximum(knorm, 1e-12)
    mnorm = jnp.sqrt(jnp.sum(mem * mem, axis=1, keepdims=True))
    mn = mem / jnp.maximum(mnorm, 1e-12)

    sim = lax.dot_general(kn, mn, (((1,), (1,)), ((), ())),
                          preferred_element_type=jnp.float32) * beta

    bits = lax.bitcast_convert_type(sim, jnp.uint32)
    sign = bits >> 31
    ordered = bits ^ (jnp.uint32(0x80000000) + sign * jnp.uint32(0x7FFFFFFF))

    def step(i, cand):
        test = cand | (jnp.uint32(1) << (31 - i).astype(jnp.uint32))
        cnt = jnp.sum((ordered >= test).astype(jnp.int32), axis=1,
                      keepdims=True)
        return jnp.where(cnt >= TOPK_K, test, cand)

    cand = lax.fori_loop(0, 32, step, jnp.zeros((H, 1), jnp.uint32),
                         unroll=True)
    mask = ordered >= cand

    rowmax = jnp.max(sim, axis=1, keepdims=True)
    e = jnp.where(mask, jnp.exp(sim - rowmax), 0.0)
    w = e / jnp.sum(e, axis=1, keepdims=True)
    w_ref[i] = w

    rph = lax.dot_general(w, mem, (((1,), (0,)), ((), ())),
                          preferred_element_type=jnp.float32)

    merged = bm_ref[...]
    for h in range(H):
        merged = merged + lax.dot_general(
            rph[h:h + 1], wm_ref[h], (((1,), (1,)), ((), ())),
            preferred_element_type=jnp.float32)

    mu = jnp.mean(merged, axis=1, keepdims=True)
    var = jnp.mean((merged - mu) ** 2, axis=1, keepdims=True)
    out_ref[i] = ((merged - mu) / jnp.sqrt(var + 1e-5)) * gam_ref[...] \
        + lnb_ref[...]


def _fused_body(mem_ref, keys_ref, beta_ref, wm_ref, bm_ref, gam_ref,
                lnb_ref, out_ref, w_ref):
    for i in range(mem_ref.shape[0]):
        _one_batch(i, mem_ref, keys_ref, beta_ref, wm_ref, bm_ref, gam_ref,
                   lnb_ref, out_ref, w_ref)


@jax.jit
def kernel(memory, read_keys, beta, W_merge, b_merge, ln_gamma, ln_beta):
    B, N, D = memory.shape
    H = read_keys.shape[1]
    wm_r = W_merge.reshape(D, H, D).transpose(1, 0, 2)
    beta_r = beta[..., None]
    bm2 = b_merge[None, :]
    gam2 = ln_gamma[None, :]
    lnb2 = ln_beta[None, :]

    out = pl.pallas_call(
        _fused_body,
        grid=(B,),
        in_specs=[
            pl.BlockSpec((1, N, D), lambda b: (b, 0, 0)),
            pl.BlockSpec((1, H, D), lambda b: (b, 0, 0)),
            pl.BlockSpec((1, H, 1), lambda b: (b, 0, 0)),
            pl.BlockSpec((H, D, D), lambda b: (0, 0, 0)),
            pl.BlockSpec((1, D), lambda b: (0, 0)),
            pl.BlockSpec((1, D), lambda b: (0, 0)),
            pl.BlockSpec((1, D), lambda b: (0, 0)),
        ],
        out_specs=[
            pl.BlockSpec((1, 1, D), lambda b: (b, 0, 0)),
            pl.BlockSpec((1, H, N), lambda b: (b, 0, 0)),
        ],
        out_shape=[
            jax.ShapeDtypeStruct((B, 1, D), jnp.float32),
            jax.ShapeDtypeStruct((B, H, N), jnp.float32),
        ],
        compiler_params=pltpu.CompilerParams(
            dimension_semantics=("arbitrary",),
            vmem_limit_bytes=100 * 1024 * 1024,
        ),
    )(memory, read_keys, beta_r, wm_r, bm2, gam2, lnb2)
    return out[0][:, 0, :], out[1]
